# baseline (speedup 1.0000x reference)
import functools

import jax
import jax.numpy as jnp
from jax import lax
from jax.experimental import pallas as pl
from jax.experimental.pallas import tpu as pltpu
from jax.experimental.pallas import tpu_sc as plsc

_BATCH = 16384
_D = 32
_PK = 4
_NC = 2
_NS = 16
_NW = _NC * _NS
_BPW = _BATCH // _NW
_LPW = _BPW // _PK


_FPW = _BPW * _D


def _sc_gather_flat_body(u_ids, i_ids, ut, it, u_out, i_out,
                         sid_u, sid_i, rows_u, rows_i, sem):
    wid = lax.axis_index("s") * _NC + lax.axis_index("c")
    base = wid * _BPW
    pltpu.sync_copy(u_ids.at[pl.ds(base, _BPW)], sid_u)
    pltpu.sync_copy(i_ids.at[pl.ds(base, _BPW)], sid_i)

    def group_body(g, _):
        vu = sid_u[pl.ds(g * 16, 16)] * _D
        vi = sid_i[pl.ds(g * 16, 16)] * _D
        for l in range(16):
            off = pl.multiple_of((g * 16 + l) * _D, _D)
            pltpu.async_copy(ut.at[pl.ds(pl.multiple_of(vu[l], _D), _D)],
                             rows_u.at[pl.ds(off, _D)], sem)
            pltpu.async_copy(it.at[pl.ds(pl.multiple_of(vi[l], _D), _D)],
                             rows_i.at[pl.ds(off, _D)], sem)
        return 0

    lax.fori_loop(0, _BPW // 16, group_body, 0)

    def drain_body(r, _):
        pltpu.make_async_copy(ut.at[pl.ds(0, _D)],
                              rows_u.at[pl.ds(0, _D)], sem).wait()
        pltpu.make_async_copy(it.at[pl.ds(0, _D)],
                              rows_i.at[pl.ds(0, _D)], sem).wait()
        return 0

    lax.fori_loop(0, _BPW, drain_body, 0)
    pltpu.sync_copy(rows_u, u_out.at[pl.ds(wid * _FPW, _FPW)])
    pltpu.sync_copy(rows_i, i_out.at[pl.ds(wid * _FPW, _FPW)])


def _sc_gather_body(u_ids, i_ids, ut, it, u_out, i_out,
                    sid_u, sid_i, rows, pk_u, pk_i, sem):
    wid = lax.axis_index("s") * _NC + lax.axis_index("c")
    base = wid * _BPW
    pltpu.sync_copy(u_ids.at[pl.ds(base, _BPW)], sid_u)
    pltpu.sync_copy(i_ids.at[pl.ds(base, _BPW)], sid_i)

    for tbl, sid, pk in ((ut, sid_u, pk_u), (it, sid_i, pk_i)):

        def group_body(g, _):
            v = sid[pl.ds(g * 16, 16)]
            for l in range(16):
                pltpu.async_copy(tbl.at[pl.ds(v[l], 1)],
                                 rows.at[pl.ds(g * 16 + l, 1)], sem)
            return 0

        lax.fori_loop(0, _BPW // 16, group_body, 0)

        def drain_body(r, _):
            pltpu.make_async_copy(tbl.at[pl.ds(0, 1)],
                                  rows.at[pl.ds(0, 1)], sem).wait()
            return 0

        lax.fori_loop(0, _BPW, drain_body, 0)

        def pack_body(r, _):
            ln = r // _PK
            cs = (r % _PK) * _D
            for k in range(_D // 16):
                pk[ln, pl.ds(cs + k * 16, 16)] = rows[r, pl.ds(k * 16, 16)]
            return 0

        lax.fori_loop(0, _BPW, pack_body, 0)

    pltpu.sync_copy(pk_u, u_out.at[pl.ds(wid * _LPW, _LPW)])
    pltpu.sync_copy(pk_i, i_out.at[pl.ds(wid * _LPW, _LPW)])


def _mlp_body(u_ref, i_ref, k0a, k0b, b0, k1, b1, k2, b2, k3, b3, k4, b4,
              out_ref):
    x = jnp.dot(u_ref[...], k0a[...], preferred_element_type=jnp.float32)
    x = x + jnp.dot(i_ref[...], k0b[...], preferred_element_type=jnp.float32)
    h = jnp.maximum(x + b0[...], 0.0)
    h = jnp.maximum(
        jnp.dot(h, k1[...], preferred_element_type=jnp.float32) + b1[...], 0.0)
    h = jnp.maximum(
        jnp.dot(h, k2[...], preferred_element_type=jnp.float32) + b2[...], 0.0)
    h = jnp.maximum(
        jnp.dot(h, k3[...], preferred_element_type=jnp.float32) + b3[...], 0.0)
    out_ref[...] = (
        jnp.dot(h, k4[...], preferred_element_type=jnp.float32) + b4[...])


def kernel(U_ids, I_ids, user_table, item_table,
           W0, b0, W1, b1, W2, b2, W3, b3, W4, b4):
    u_ids = U_ids.astype(jnp.int32)
    i_ids = I_ids.astype(jnp.int32)

    ker = jnp.eye(_PK * _D, dtype=jnp.float32).reshape(_PK, _D, _PK * _D)
    pack = lambda t: lax.conv_general_dilated(
        t.reshape(1, 1000000, _D), ker, window_strides=(_PK,),
        padding="VALID",
        dimension_numbers=("NWC", "WIO", "NWC")).reshape(-1)
    ut = pack(user_table)
    it = pack(item_table)

    sc = functools.partial(
        pl.kernel,
        mesh=plsc.VectorSubcoreMesh(core_axis_name="c", subcore_axis_name="s"),
        out_type=[
            jax.ShapeDtypeStruct((_BATCH * _D,), jnp.float32),
            jax.ShapeDtypeStruct((_BATCH * _D,), jnp.float32),
        ],
        scratch_types=[
            pltpu.VMEM((_BPW,), jnp.int32),
            pltpu.VMEM((_BPW,), jnp.int32),
            pltpu.VMEM((_FPW,), jnp.float32),
            pltpu.VMEM((_FPW,), jnp.float32),
            pltpu.SemaphoreType.DMA,
        ],
    )(_sc_gather_flat_body)
    u_flat, i_flat = sc(u_ids, i_ids, ut, it)
    u_rows = u_flat.reshape(_BATCH // _PK, _PK * _D)
    i_rows = i_flat.reshape(_BATCH // _PK, _PK * _D)

    eye = jnp.eye(_PK, dtype=jnp.float32)
    kr = lambda w: jnp.kron(eye, w)
    tl = lambda b: jnp.tile(b, _PK).reshape(1, -1)
    out = pl.pallas_call(
        _mlp_body,
        out_shape=jax.ShapeDtypeStruct((_BATCH // _PK, _PK), jnp.float32),
    )(u_rows, i_rows,
      kr(W0[:_D]), kr(W0[_D:]), tl(b0),
      kr(W1), tl(b1),
      kr(W2), tl(b2),
      kr(W3), tl(b3),
      kr(W4), tl(b4))
    return out.reshape(_BATCH, 1)


# speedup vs baseline: 5.8245x; 5.8245x over previous
import functools

import jax
import jax.numpy as jnp
from jax import lax
from jax.experimental import pallas as pl
from jax.experimental.pallas import tpu as pltpu
from jax.experimental.pallas import tpu_sc as plsc

_BATCH = 16384
_D = 32
_PK = 4
_NC = 2
_NS = 16
_NW = _NC * _NS
_BPW = _BATCH // _NW
_LPW = _BPW // _PK


def _sc_gather_body(u_ids, i_ids, ut, it, u_out, i_out,
                    sid_u, sid_i, rows, pk_u, pk_i, sem):
    wid = lax.axis_index("s") * _NC + lax.axis_index("c")
    base = wid * _BPW
    pltpu.sync_copy(u_ids.at[pl.ds(base, _BPW)], sid_u)
    pltpu.sync_copy(i_ids.at[pl.ds(base, _BPW)], sid_i)

    for tbl, sid, pk in ((ut, sid_u, pk_u), (it, sid_i, pk_i)):

        def group_body(g, _):
            v = sid[pl.ds(g * 16, 16)]
            for l in range(16):
                pltpu.async_copy(tbl.at[pl.ds(v[l], 1)],
                                 rows.at[pl.ds(g * 16 + l, 1)], sem)
            return 0

        lax.fori_loop(0, _BPW // 16, group_body, 0)

        def drain_body(r, _):
            pltpu.make_async_copy(tbl.at[pl.ds(0, 1)],
                                  rows.at[pl.ds(0, 1)], sem).wait()
            return 0

        lax.fori_loop(0, _BPW, drain_body, 0)

        def pack_body(r, _):
            ln = r // _PK
            cs = (r % _PK) * _D
            for k in range(_D // 16):
                pk[ln, pl.ds(cs + k * 16, 16)] = rows[r, pl.ds(k * 16, 16)]
            return 0

        lax.fori_loop(0, _BPW, pack_body, 0)

    pltpu.sync_copy(pk_u, u_out.at[pl.ds(wid * _LPW, _LPW)])
    pltpu.sync_copy(pk_i, i_out.at[pl.ds(wid * _LPW, _LPW)])


def _mlp_body(u_ref, i_ref, k0a, k0b, b0, k1, b1, k2, b2, k3, b3, k4, b4,
              out_ref):
    x = jnp.dot(u_ref[...], k0a[...], preferred_element_type=jnp.float32)
    x = x + jnp.dot(i_ref[...], k0b[...], preferred_element_type=jnp.float32)
    h = jnp.maximum(x + b0[...], 0.0)
    h = jnp.maximum(
        jnp.dot(h, k1[...], preferred_element_type=jnp.float32) + b1[...], 0.0)
    h = jnp.maximum(
        jnp.dot(h, k2[...], preferred_element_type=jnp.float32) + b2[...], 0.0)
    h = jnp.maximum(
        jnp.dot(h, k3[...], preferred_element_type=jnp.float32) + b3[...], 0.0)
    out_ref[...] = (
        jnp.dot(h, k4[...], preferred_element_type=jnp.float32) + b4[...])


def kernel(U_ids, I_ids, user_table, item_table,
           W0, b0, W1, b1, W2, b2, W3, b3, W4, b4):
    u_ids = U_ids.astype(jnp.int32)
    i_ids = I_ids.astype(jnp.int32)

    sc = functools.partial(
        pl.kernel,
        mesh=plsc.VectorSubcoreMesh(core_axis_name="c", subcore_axis_name="s"),
        out_type=[
            jax.ShapeDtypeStruct((_BATCH // _PK, _PK * _D), jnp.float32),
            jax.ShapeDtypeStruct((_BATCH // _PK, _PK * _D), jnp.float32),
        ],
        scratch_types=[
            pltpu.VMEM((_BPW,), jnp.int32),
            pltpu.VMEM((_BPW,), jnp.int32),
            pltpu.VMEM((_BPW, _D), jnp.float32),
            pltpu.VMEM((_LPW, _PK * _D), jnp.float32),
            pltpu.VMEM((_LPW, _PK * _D), jnp.float32),
            pltpu.SemaphoreType.DMA,
        ],
    )(_sc_gather_body)
    u_rows, i_rows = sc(u_ids, i_ids, user_table, item_table)

    eye = jnp.eye(_PK, dtype=jnp.float32)
    kr = lambda w: jnp.kron(eye, w)
    tl = lambda b: jnp.tile(b, _PK).reshape(1, -1)
    out = pl.pallas_call(
        _mlp_body,
        out_shape=jax.ShapeDtypeStruct((_BATCH // _PK, _PK), jnp.float32),
    )(u_rows, i_rows,
      kr(W0[:_D]), kr(W0[_D:]), tl(b0),
      kr(W1), tl(b1),
      kr(W2), tl(b2),
      kr(W3), tl(b3),
      kr(W4), tl(b4))
    return out.reshape(_BATCH, 1)


# speedup vs baseline: 5.9004x; 1.0130x over previous
import functools

import jax
import jax.numpy as jnp
from jax import lax
from jax.experimental import pallas as pl
from jax.experimental.pallas import tpu as pltpu
from jax.experimental.pallas import tpu_sc as plsc

_BATCH = 16384
_D = 32
_PK = 4
_NC = 2
_NS = 16
_NW = _NC * _NS
_BPW = _BATCH // _NW
_LPW = _BPW // _PK


def _sc_gather_body(ids, tbl, out, sid, rows, pk, sem):
    wid = lax.axis_index("s") * _NC + lax.axis_index("c")
    base = wid * _BPW
    pltpu.sync_copy(ids.at[pl.ds(base, _BPW)], sid)

    def group_body(g, _):
        v = sid[pl.ds(g * 16, 16)]
        for l in range(16):
            pltpu.async_copy(tbl.at[pl.ds(v[l], 1)],
                             rows.at[pl.ds(g * 16 + l, 1)], sem)
        return 0

    lax.fori_loop(0, _BPW // 16, group_body, 0)

    def drain_body(r, _):
        pltpu.make_async_copy(tbl.at[pl.ds(0, 1)],
                              rows.at[pl.ds(0, 1)], sem).wait()
        return 0

    lax.fori_loop(0, _BPW, drain_body, 0)

    def pack_body(r, _):
        ln = r // _PK
        cs = (r % _PK) * _D
        for k in range(_D // 16):
            pk[ln, pl.ds(cs + k * 16, 16)] = rows[r, pl.ds(k * 16, 16)]
        return 0

    lax.fori_loop(0, _BPW, pack_body, 0)
    pltpu.sync_copy(pk, out.at[pl.ds(wid * _LPW, _LPW)])


def _mlp_body(u_ref, i_ref, k0a, k0b, b0, k1, b1, k2, b2, k3, b3, k4, b4,
              out_ref):
    x = jnp.dot(u_ref[...], k0a[...], preferred_element_type=jnp.float32)
    x = x + jnp.dot(i_ref[...], k0b[...], preferred_element_type=jnp.float32)
    h = jnp.maximum(x + b0[...], 0.0)
    h = jnp.maximum(
        jnp.dot(h, k1[...], preferred_element_type=jnp.float32) + b1[...], 0.0)
    h = jnp.maximum(
        jnp.dot(h, k2[...], preferred_element_type=jnp.float32) + b2[...], 0.0)
    h = jnp.maximum(
        jnp.dot(h, k3[...], preferred_element_type=jnp.float32) + b3[...], 0.0)
    out_ref[...] = (
        jnp.dot(h, k4[...], preferred_element_type=jnp.float32) + b4[...])


def kernel(U_ids, I_ids, user_table, item_table,
           W0, b0, W1, b1, W2, b2, W3, b3, W4, b4):
    u_ids = U_ids.astype(jnp.int32)
    i_ids = I_ids.astype(jnp.int32)

    sc = functools.partial(
        pl.kernel,
        mesh=plsc.VectorSubcoreMesh(core_axis_name="c", subcore_axis_name="s"),
        out_type=jax.ShapeDtypeStruct((_BATCH // _PK, _PK * _D), jnp.float32),
        scratch_types=[
            pltpu.VMEM((_BPW,), jnp.int32),
            pltpu.VMEM((_BPW, _D), jnp.float32),
            pltpu.VMEM((_LPW, _PK * _D), jnp.float32),
            pltpu.SemaphoreType.DMA,
        ],
    )(_sc_gather_body)
    u_rows = sc(u_ids, user_table)
    i_rows = sc(i_ids, item_table)

    eye = jnp.eye(_PK, dtype=jnp.float32)
    kr = lambda w: jnp.kron(eye, w)
    tl = lambda b: jnp.tile(b, _PK).reshape(1, -1)
    out = pl.pallas_call(
        _mlp_body,
        out_shape=jax.ShapeDtypeStruct((_BATCH // _PK, _PK), jnp.float32),
    )(u_rows, i_rows,
      kr(W0[:_D]), kr(W0[_D:]), tl(b0),
      kr(W1), tl(b1),
      kr(W2), tl(b2),
      kr(W3), tl(b3),
      kr(W4), tl(b4))
    return out.reshape(_BATCH, 1)
